# baseline (speedup 1.0000x reference)
import functools
import math

import jax
import jax.numpy as jnp
from jax import lax
from jax.experimental import pallas as pl
from jax.experimental.pallas import tpu as pltpu
from jax.experimental.pallas import tpu_sc as plsc


def _make_sc_kernel(B, S, V, D):
    info = plsc.get_sparse_core_info()
    NC, NS, L = info.num_cores, info.num_subcores, info.num_lanes
    NW = NC * NS
    assert S % NW == 0
    pos_per_w = S // NW
    P = 32
    n_chunks = pos_per_w // P
    scale = jnp.float32(math.sqrt(D))
    vecs_per_row = D // L

    mesh = plsc.VectorSubcoreMesh(core_axis_name="c", subcore_axis_name="s")

    @functools.partial(
        pl.kernel,
        out_type=jax.ShapeDtypeStruct((B, S, D), jnp.float32),
        mesh=mesh,
        scratch_types=[
            pltpu.VMEM((B, pos_per_w), jnp.int32),
            pltpu.VMEM((P, D), jnp.float32),
            pltpu.VMEM((P, D), jnp.float32),
            pltpu.SemaphoreType.DMA,
        ],
    )
    def emb_kernel(x_hbm, table_hbm, pos_hbm, out_hbm, idx_v, pos_v, rows_v, sem):
        wid = lax.axis_index("s") * NC + lax.axis_index("c")
        base = wid * pos_per_w
        for b in range(B):
            pltpu.sync_copy(x_hbm.at[b, pl.ds(base, pos_per_w)], idx_v.at[b])
        for ch in range(n_chunks):
            p0 = base + ch * P
            pltpu.sync_copy(pos_hbm.at[pl.ds(p0, P)], pos_v)
            for b in range(B):
                pltpu.async_copy(
                    table_hbm.at[idx_v.at[b, pl.ds(ch * P, P)]], rows_v, sem
                ).wait()

                def row_body(r, _):
                    for cb in range(vecs_per_row):
                        sl = (r, pl.ds(cb * L, L))
                        rows_v[sl] = rows_v[sl] * scale + pos_v[sl]
                    return 0

                lax.fori_loop(0, P, row_body, 0)
                pltpu.sync_copy(rows_v, out_hbm.at[b, pl.ds(p0, P)])

    return emb_kernel


def kernel(x, emb_table, pos_enc):
    B, S = x.shape
    V, D = emb_table.shape
    x = x.astype(jnp.int32)
    emb = _make_sc_kernel(B, S, V, D)
    return emb(x, emb_table, pos_enc)


# speedup vs baseline: 1.3639x; 1.3639x over previous
import functools
import math

import jax
import jax.numpy as jnp
from jax import lax
from jax.experimental import pallas as pl
from jax.experimental.pallas import tpu as pltpu
from jax.experimental.pallas import tpu_sc as plsc


def _make_sc_kernel(B, S, V, D):
    info = plsc.get_sparse_core_info()
    NC, NS, L = info.num_cores, info.num_subcores, info.num_lanes
    NW = NC * NS
    assert S % NW == 0
    pos_per_w = S // NW
    P = 16
    n_chunks = pos_per_w // P
    n_items = n_chunks * B
    scale = jnp.float32(math.sqrt(D))
    vecs_per_row = D // L

    mesh = plsc.VectorSubcoreMesh(core_axis_name="c", subcore_axis_name="s")

    @functools.partial(
        pl.kernel,
        out_type=jax.ShapeDtypeStruct((B, S, D), jnp.float32),
        mesh=mesh,
        scratch_types=[
            pltpu.VMEM((B, pos_per_w), jnp.int32),
            pltpu.VMEM((P, D), jnp.float32),
            pltpu.VMEM((P, D), jnp.float32),
            pltpu.VMEM((P, D), jnp.float32),
            pltpu.VMEM((P, D), jnp.float32),
            pltpu.VMEM((P, D), jnp.float32),
            pltpu.VMEM((P, D), jnp.float32),
            pltpu.SemaphoreType.DMA,
            pltpu.SemaphoreType.DMA,
            pltpu.SemaphoreType.DMA,
            pltpu.SemaphoreType.DMA,
            pltpu.SemaphoreType.DMA,
            pltpu.SemaphoreType.DMA,
        ],
    )
    def emb_kernel(x_hbm, table_hbm, pos_hbm, out_hbm,
                   idx_v, rows0, rows1, outb0, outb1, posb0, posb1,
                   gsem0, gsem1, osem0, osem1, psem0, psem1):
        wid = lax.axis_index("s") * NC + lax.axis_index("c")
        base = wid * pos_per_w
        rows = (rows0, rows1)
        outs = (outb0, outb1)
        poss = (posb0, posb1)
        gsems = (gsem0, gsem1)
        osems = (osem0, osem1)
        psems = (psem0, psem1)

        for b in range(B):
            pltpu.sync_copy(x_hbm.at[b, pl.ds(base, pos_per_w)], idx_v.at[b])

        def start_pos(ch):
            return pltpu.async_copy(
                pos_hbm.at[pl.ds(base + ch * P, P)], poss[ch % 2], psems[ch % 2])

        def start_gather(i):
            ch, b = divmod(i, B)
            return pltpu.async_copy(
                table_hbm.at[idx_v.at[b, pl.ds(ch * P, P)]], rows[i % 2],
                gsems[i % 2])

        pos_copies = {0: start_pos(0), 1: start_pos(1)}
        gather_copies = {0: start_gather(0), 1: start_gather(1)}
        out_copies = {}

        for i in range(n_items):
            ch, b = divmod(i, B)
            if b == 0:
                pos_copies[ch].wait()
            gather_copies[i].wait()
            if i >= 2:
                out_copies[i - 2].wait()
            rbuf, obuf, pbuf = rows[i % 2], outs[i % 2], poss[ch % 2]

            def row_body(r, _, rbuf=rbuf, obuf=obuf, pbuf=pbuf):
                for cb in range(vecs_per_row):
                    sl = (r, pl.ds(cb * L, L))
                    obuf[sl] = rbuf[sl] * scale + pbuf[sl]
                return 0

            lax.fori_loop(0, P, row_body, 0)
            out_copies[i] = pltpu.async_copy(
                obuf, out_hbm.at[b, pl.ds(base + ch * P, P)], osems[i % 2])
            if i + 2 < n_items:
                gather_copies[i + 2] = start_gather(i + 2)
            if b == B - 1 and ch + 2 < n_chunks:
                pos_copies[ch + 2] = start_pos(ch + 2)

        out_copies[n_items - 2].wait()
        out_copies[n_items - 1].wait()

    return emb_kernel


def kernel(x, emb_table, pos_enc):
    B, S = x.shape
    V, D = emb_table.shape
    x = x.astype(jnp.int32)
    emb = _make_sc_kernel(B, S, V, D)
    return emb(x, emb_table, pos_enc)


# speedup vs baseline: 1.5254x; 1.1184x over previous
import functools
import math

import jax
import jax.numpy as jnp
from jax import lax
from jax.experimental import pallas as pl
from jax.experimental.pallas import tpu as pltpu
from jax.experimental.pallas import tpu_sc as plsc


def _make_sc_kernel(B, S, V, D):
    info = plsc.get_sparse_core_info()
    NC, NS, L = info.num_cores, info.num_subcores, info.num_lanes
    NW = NC * NS
    assert S % NW == 0
    pos_per_w = S // NW
    P = 8
    n_chunks = pos_per_w // P
    NBUF = 3
    scale = jnp.float32(math.sqrt(D))
    vecs_per_row = D // L

    mesh = plsc.VectorSubcoreMesh(core_axis_name="c", subcore_axis_name="s")

    @functools.partial(
        pl.kernel,
        out_type=jax.ShapeDtypeStruct((B, S, D), jnp.float32),
        mesh=mesh,
        scratch_types=[
            pltpu.VMEM((B, pos_per_w), jnp.int32),
            pltpu.VMEM((NBUF, B, P, D), jnp.float32),
            pltpu.VMEM((2, P, D), jnp.float32),
            pltpu.SemaphoreType.DMA,
            pltpu.SemaphoreType.DMA,
            pltpu.SemaphoreType.DMA,
            pltpu.SemaphoreType.DMA,
            pltpu.SemaphoreType.DMA,
            pltpu.SemaphoreType.DMA,
            pltpu.SemaphoreType.DMA,
            pltpu.SemaphoreType.DMA,
        ],
    )
    def emb_kernel(x_hbm, table_hbm, pos_hbm, out_hbm, idx_v, rows_v, pos_v,
                   gsem0, gsem1, gsem2, osem0, osem1, osem2, psem0, psem1):
        wid = lax.axis_index("s") * NC + lax.axis_index("c")
        base = wid * pos_per_w
        gsems = (gsem0, gsem1, gsem2)
        osems = (osem0, osem1, osem2)
        psems = (psem0, psem1)

        for b in range(B):
            pltpu.sync_copy(x_hbm.at[b, pl.ds(base, pos_per_w)], idx_v.at[b])

        def start_pos(ch):
            return pltpu.async_copy(
                pos_hbm.at[pl.ds(base + ch * P, P)], pos_v.at[ch % 2],
                psems[ch % 2])

        def start_gathers(ch):
            par = ch % NBUF
            return [
                pltpu.async_copy(
                    table_hbm.at[idx_v.at[b, pl.ds(ch * P, P)]],
                    rows_v.at[par, b], gsems[par])
                for b in range(B)
            ]

        pos_copies = {0: start_pos(0), 1: start_pos(1)}
        gather_copies = {0: start_gathers(0), 1: start_gathers(1)}
        wb_copies = {}

        for ch in range(n_chunks):
            par = ch % NBUF
            pos_copies[ch].wait()
            for c in gather_copies[ch]:
                c.wait()

            pi = ch % 2

            def body(i, _, par=par, pi=pi):
                r = i // 8
                g = i % 8
                for k in range(8):
                    cb = g * 8 + k
                    sl = pl.ds(cb * L, L)
                    pv = pos_v[pi, r, sl]
                    for b in range(B):
                        rows_v[par, b, r, sl] = rows_v[par, b, r, sl] * scale + pv
                return 0

            lax.fori_loop(0, P * 8, body, 0)

            wb_copies[ch] = [
                pltpu.async_copy(
                    rows_v.at[par, b], out_hbm.at[b, pl.ds(base + ch * P, P)],
                    osems[par])
                for b in range(B)
            ]
            if ch + 2 < n_chunks:
                pos_copies[ch + 2] = start_pos(ch + 2)
            if ch >= 1:
                for c in wb_copies[ch - 1]:
                    c.wait()
            if ch + 2 < n_chunks:
                gather_copies[ch + 2] = start_gathers(ch + 2)

        for c in wb_copies[n_chunks - 1]:
            c.wait()

    return emb_kernel


def kernel(x, emb_table, pos_enc):
    B, S = x.shape
    V, D = emb_table.shape
    x = x.astype(jnp.int32)
    emb = _make_sc_kernel(B, S, V, D)
    return emb(x, emb_table, pos_enc)


# speedup vs baseline: 1.5967x; 1.0467x over previous
import functools
import math

import jax
import jax.numpy as jnp
from jax import lax
from jax.experimental import pallas as pl
from jax.experimental.pallas import tpu as pltpu
from jax.experimental.pallas import tpu_sc as plsc


def _make_sc_kernel(B, S, V, D):
    info = plsc.get_sparse_core_info()
    NC, NS, L = info.num_cores, info.num_subcores, info.num_lanes
    NW = NC * NS
    assert S % NW == 0
    pos_per_w = S // NW
    P = 8
    n_chunks = pos_per_w // P
    NBUF = 3
    scale = jnp.float32(math.sqrt(D))
    vecs_per_row = D // L

    mesh = plsc.VectorSubcoreMesh(core_axis_name="c", subcore_axis_name="s")

    @functools.partial(
        pl.kernel,
        out_type=jax.ShapeDtypeStruct((B, S, D), jnp.float32),
        mesh=mesh,
        scratch_types=[
            pltpu.VMEM((B, pos_per_w), jnp.int32),
            pltpu.VMEM((NBUF, B, P, D), jnp.float32),
            pltpu.VMEM((2, P, D), jnp.float32),
            pltpu.SemaphoreType.DMA,
            pltpu.SemaphoreType.DMA,
            pltpu.SemaphoreType.DMA,
            pltpu.SemaphoreType.DMA,
            pltpu.SemaphoreType.DMA,
            pltpu.SemaphoreType.DMA,
            pltpu.SemaphoreType.DMA,
            pltpu.SemaphoreType.DMA,
        ],
    )
    def emb_kernel(x_hbm, table_hbm, pos_hbm, out_hbm, idx_v, rows_v, pos_v,
                   gsem0, gsem1, gsem2, osem0, osem1, osem2, psem0, psem1):
        wid = lax.axis_index("s") * NC + lax.axis_index("c")
        base = wid * pos_per_w
        gsems = (gsem0, gsem1, gsem2)
        osems = (osem0, osem1, osem2)
        psems = (psem0, psem1)

        for b in range(B):
            pltpu.sync_copy(x_hbm.at[b, pl.ds(base, pos_per_w)], idx_v.at[b])

        def start_pos(ch):
            return pltpu.async_copy(
                pos_hbm.at[pl.ds(base + ch * P, P)], pos_v.at[ch % 2],
                psems[ch % 2])

        def start_gathers(ch):
            par = ch % NBUF
            return [
                pltpu.async_copy(
                    table_hbm.at[idx_v.at[b, pl.ds(ch * P, P)]],
                    rows_v.at[par, b], gsems[par])
                for b in range(B)
            ]

        pos_copies = {0: start_pos(0), 1: start_pos(1)}
        gather_copies = {0: start_gathers(0), 1: start_gathers(1)}
        wb_copies = {}

        for ch in range(n_chunks):
            par = ch % NBUF
            pos_copies[ch].wait()
            for c in gather_copies[ch]:
                c.wait()

            pi = ch % 2

            @plsc.parallel_loop(0, P * 8)
            def body(i, par=par, pi=pi):
                r = i // 8
                g = i % 8
                for k in range(8):
                    cb = g * 8 + k
                    sl = pl.ds(cb * L, L)
                    pv = pos_v[pi, r, sl]
                    for b in range(B):
                        rows_v[par, b, r, sl] = rows_v[par, b, r, sl] * scale + pv

            wb_copies[ch] = [
                pltpu.async_copy(
                    rows_v.at[par, b], out_hbm.at[b, pl.ds(base + ch * P, P)],
                    osems[par])
                for b in range(B)
            ]
            if ch + 2 < n_chunks:
                pos_copies[ch + 2] = start_pos(ch + 2)
            if ch >= 1:
                for c in wb_copies[ch - 1]:
                    c.wait()
            if ch + 2 < n_chunks:
                gather_copies[ch + 2] = start_gathers(ch + 2)

        for c in wb_copies[n_chunks - 1]:
            c.wait()

    return emb_kernel


def kernel(x, emb_table, pos_enc):
    B, S = x.shape
    V, D = emb_table.shape
    x = x.astype(jnp.int32)
    emb = _make_sc_kernel(B, S, V, D)
    return emb(x, emb_table, pos_enc)


# speedup vs baseline: 1.6338x; 1.0233x over previous
import functools
import math

import jax
import jax.numpy as jnp
from jax import lax
from jax.experimental import pallas as pl
from jax.experimental.pallas import tpu as pltpu
from jax.experimental.pallas import tpu_sc as plsc


def _make_sc_kernel(B, S, V, D):
    info = plsc.get_sparse_core_info()
    NC, NS, L = info.num_cores, info.num_subcores, info.num_lanes
    NW = NC * NS
    assert S % NW == 0
    pos_per_w = S // NW
    P = 8
    n_chunks = pos_per_w // P
    NBUF = 3
    scale = jnp.float32(math.sqrt(D))
    vecs_per_row = D // L

    mesh = plsc.VectorSubcoreMesh(core_axis_name="c", subcore_axis_name="s")

    @functools.partial(
        pl.kernel,
        out_type=jax.ShapeDtypeStruct((B, S, D), jnp.float32),
        mesh=mesh,
        scratch_types=[
            pltpu.VMEM((B, pos_per_w), jnp.int32),
            pltpu.VMEM((NBUF, B, P, D), jnp.float32),
            pltpu.VMEM((2, P, D), jnp.float32),
            pltpu.SemaphoreType.DMA,
            pltpu.SemaphoreType.DMA,
            pltpu.SemaphoreType.DMA,
            pltpu.SemaphoreType.DMA,
            pltpu.SemaphoreType.DMA,
            pltpu.SemaphoreType.DMA,
            pltpu.SemaphoreType.DMA,
            pltpu.SemaphoreType.DMA,
            pltpu.SemaphoreType.DMA,
        ],
    )
    def emb_kernel(x_hbm, table_hbm, pos_hbm, out_hbm, idx_v, rows_v, pos_v,
                   gsem0, gsem1, gsem2, osem0, osem1, osem2, psem0, psem1,
                   isem):
        wid = lax.axis_index("s") * NC + lax.axis_index("c")
        base = wid * pos_per_w
        gsems = (gsem0, gsem1, gsem2)
        osems = (osem0, osem1, osem2)
        psems = (psem0, psem1)

        idx_copies = [
            pltpu.async_copy(x_hbm.at[b, pl.ds(base, pos_per_w)], idx_v.at[b],
                             isem)
            for b in range(B)
        ]

        def start_pos(ch):
            return pltpu.async_copy(
                pos_hbm.at[pl.ds(base + ch * P, P)], pos_v.at[ch % 2],
                psems[ch % 2])

        def start_gathers(ch):
            par = ch % NBUF
            return [
                pltpu.async_copy(
                    table_hbm.at[idx_v.at[b, pl.ds(ch * P, P)]],
                    rows_v.at[par, b], gsems[par])
                for b in range(B)
            ]

        pos_copies = {0: start_pos(0), 1: start_pos(1)}
        for c in idx_copies:
            c.wait()
        gather_copies = {0: start_gathers(0), 1: start_gathers(1)}
        wb_copies = {}

        for ch in range(n_chunks):
            par = ch % NBUF
            pos_copies[ch].wait()
            for c in gather_copies[ch]:
                c.wait()

            pi = ch % 2

            @plsc.parallel_loop(0, P * 8)
            def body(i, par=par, pi=pi):
                r = i // 8
                g = i % 8
                for k in range(8):
                    cb = g * 8 + k
                    sl = pl.ds(cb * L, L)
                    pv = pos_v[pi, r, sl]
                    for b in range(B):
                        rows_v[par, b, r, sl] = rows_v[par, b, r, sl] * scale + pv

            wb_copies[ch] = [
                pltpu.async_copy(
                    rows_v.at[par, b], out_hbm.at[b, pl.ds(base + ch * P, P)],
                    osems[par])
                for b in range(B)
            ]
            if ch + 2 < n_chunks:
                pos_copies[ch + 2] = start_pos(ch + 2)
            if ch >= 1:
                for c in wb_copies[ch - 1]:
                    c.wait()
            if ch + 2 < n_chunks:
                gather_copies[ch + 2] = start_gathers(ch + 2)

        for c in wb_copies[n_chunks - 1]:
            c.wait()

    return emb_kernel


def kernel(x, emb_table, pos_enc):
    B, S = x.shape
    V, D = emb_table.shape
    x = x.astype(jnp.int32)
    emb = _make_sc_kernel(B, S, V, D)
    return emb(x, emb_table, pos_enc)
